# Initial kernel scaffold; baseline (speedup 1.0000x reference)
import jax
import jax.numpy as jnp
from jax.experimental import pallas as pl


def kernel(q, k, v, Wq, bq, Wk, bk, Wv, bv):
    raise NotImplementedError("write your pallas kernel here")


# speedup vs baseline: 9.2539x; 9.2539x over previous
import math
import functools

import jax
import jax.numpy as jnp
from jax.experimental import pallas as pl

_EMBED = 768
_HEADS = 12
_SEQ = 577
_HD = 64
_R1 = 404
_BATCH = 32
_BONUS = 1e4
_NSTEPS = 48

def _nonneg(x):
    return jnp.where(x < 0, jnp.exp(x), x + 1.0)


def _bf16dot(a, b, dims):
    return jax.lax.dot_general(a.astype(jnp.bfloat16), b.astype(jnp.bfloat16),
                               (dims, ((), ())),
                               preferred_element_type=jnp.float32)


def _bdot(a16, b16, dims):
    return jax.lax.dot_general(a16, b16, (dims, ((), ())),
                               preferred_element_type=jnp.float32)


def _body(q_ref, k_ref, v_ref, wq_ref, wk_ref, wv_ref, bq_ref, bk_ref,
          bv_ref, gp_ref, out_ref):
    f32 = jnp.float32
    qb = q_ref[0]
    kb = k_ref[0]
    vb = v_ref[0]

    def proj(x16, w_ref, b_ref):
        return _bdot(x16, w_ref[...], ((1,), (1,))) + b_ref[...]

    qh = proj(qb, wq_ref, bq_ref)
    kh = proj(kb, wk_ref, bk_ref)
    vh = proj(vb, wv_ref, bv_ref)

    kh16 = kh.astype(jnp.bfloat16).astype(f32)
    u16 = qh[0:1, :].astype(jnp.bfloat16).astype(f32)
    khu = kh16 * u16
    scores = jnp.concatenate(
        [jnp.sum(khu[:, h * _HD:(h + 1) * _HD], axis=1, keepdims=True)
         for h in range(_HEADS)], axis=1) * 0.125
    keys = scores + gp_ref[0]

    lo0 = jnp.min(keys, axis=0, keepdims=True) - 1.0
    hi0 = jnp.max(keys, axis=0, keepdims=True) + 1.0

    def step(_, carry):
        lo, hi = carry
        mid = 0.5 * (lo + hi)
        cnt = jnp.sum((keys >= mid).astype(f32), axis=0, keepdims=True)
        pred = cnt >= float(_R1)
        return jnp.where(pred, mid, lo), jnp.where(pred, hi, mid)

    lo, _ = jax.lax.fori_loop(0, _NSTEPS, step, (lo0, hi0))
    mask = (keys >= lo).astype(f32)

    ang = jax.lax.broadcasted_iota(jnp.int32, (_SEQ, 1), 0).astype(f32) * (
        math.pi / 2.0 / _SEQ)
    cosv = jnp.cos(ang)
    sinv = jnp.sin(ang)

    def tn(a, b):
        return _bf16dot(a, b, ((0,), (0,)))

    def nn(a, b):
        return _bf16dot(a, b, ((1,), (0,)))

    for h in range(_HEADS):
        sl = slice(h * _HD, (h + 1) * _HD)
        qnn = _nonneg(qh[:, sl])
        knn = _nonneg(kh[:, sl])
        vh_h = vh[:, sl]
        kcv = tn(knn * cosv, vh_h)
        ksv = tn(knn * sinv, vh_h)
        vals = nn(qnn * cosv, kcv) + nn(qnn * sinv, ksv)
        out_ref[0, :, sl] = vals * mask[:, h:h + 1]


@functools.partial(jax.jit, static_argnums=())
def kernel(q, k, v, Wq, bq, Wk, bk, Wv, bv):
    f32 = jnp.float32
    g = jax.random.gumbel(jax.random.key(12345), (_BATCH * _HEADS, _SEQ - 1),
                          dtype=f32)
    gt = g.reshape(_BATCH, _HEADS, _SEQ - 1).transpose(0, 2, 1)
    gpad = jnp.concatenate(
        [jnp.full((_BATCH, 1, _HEADS), _BONUS, f32), gt], axis=1)

    bspec = lambda shp: pl.BlockSpec(shp, lambda b: (0,) * len(shp))
    out = pl.pallas_call(
        _body,
        grid=(_BATCH,),
        in_specs=[
            pl.BlockSpec((1, _SEQ, _EMBED), lambda b: (b, 0, 0)),
            pl.BlockSpec((1, _SEQ, _EMBED), lambda b: (b, 0, 0)),
            pl.BlockSpec((1, _SEQ, _EMBED), lambda b: (b, 0, 0)),
            bspec((_EMBED, _EMBED)),
            bspec((_EMBED, _EMBED)),
            bspec((_EMBED, _EMBED)),
            bspec((1, _EMBED)),
            bspec((1, _EMBED)),
            bspec((1, _EMBED)),
            pl.BlockSpec((1, _SEQ, _HEADS), lambda b: (b, 0, 0)),
        ],
        out_specs=pl.BlockSpec((1, _SEQ, _EMBED), lambda b: (b, 0, 0)),
        out_shape=jax.ShapeDtypeStruct((_BATCH, _SEQ, _EMBED), f32),
    )(q.astype(jnp.bfloat16), k.astype(jnp.bfloat16), v.astype(jnp.bfloat16),
      Wq.astype(jnp.bfloat16), Wk.astype(jnp.bfloat16),
      Wv.astype(jnp.bfloat16), bq.reshape(1, -1), bk.reshape(1, -1),
      bv.reshape(1, -1), gpad)
    return out


# speedup vs baseline: 10.0768x; 1.0889x over previous
import math
import functools

import jax
import jax.numpy as jnp
from jax.experimental import pallas as pl

_EMBED = 768
_HEADS = 12
_SEQ = 577
_HD = 64
_R1 = 404
_BATCH = 32
_BONUS = 1e4
_NSTEPS = 30

def _nonneg(x):
    return jnp.where(x < 0, jnp.exp(x), x + 1.0)


def _bf16dot(a, b, dims):
    return jax.lax.dot_general(a.astype(jnp.bfloat16), b.astype(jnp.bfloat16),
                               (dims, ((), ())),
                               preferred_element_type=jnp.float32)


def _bdot(a16, b16, dims):
    return jax.lax.dot_general(a16, b16, (dims, ((), ())),
                               preferred_element_type=jnp.float32)


def _body(q_ref, k_ref, v_ref, wq_ref, wk_ref, wv_ref, bq_ref, bk_ref,
          bv_ref, gp_ref, out_ref):
    f32 = jnp.float32
    qb = q_ref[0]
    kb = k_ref[0]
    vb = v_ref[0]

    def proj(x16, w_ref, b_ref):
        return _bdot(x16, w_ref[...], ((1,), (1,))) + b_ref[...]

    qh = proj(qb, wq_ref, bq_ref)
    kh = proj(kb, wk_ref, bk_ref)
    vh = proj(vb, wv_ref, bv_ref)

    kh16 = kh.astype(jnp.bfloat16).astype(f32)
    u16 = qh[0:1, :].astype(jnp.bfloat16).astype(f32)
    khu = kh16 * u16
    scores = jnp.concatenate(
        [jnp.sum(khu[:, h * _HD:(h + 1) * _HD], axis=1, keepdims=True)
         for h in range(_HEADS)], axis=1) * 0.125
    keys = scores[1:, :] + gp_ref[0, 1:, :]

    lo0 = jnp.min(keys, axis=0, keepdims=True) - 1.0
    hi0 = jnp.max(keys, axis=0, keepdims=True) + 1.0

    def step(_, carry):
        lo, hi = carry
        mid = 0.5 * (lo + hi)
        cnt = jnp.sum((keys >= mid).astype(f32), axis=0, keepdims=True)
        pred = cnt >= float(_R1 - 1)
        return jnp.where(pred, mid, lo), jnp.where(pred, hi, mid)

    lo, _ = jax.lax.fori_loop(0, _NSTEPS, step, (lo0, hi0))
    mask = jnp.concatenate(
        [jnp.ones((1, _HEADS), f32), (keys >= lo).astype(f32)], axis=0)

    ang = jax.lax.broadcasted_iota(jnp.int32, (_SEQ, 1), 0).astype(f32) * (
        math.pi / 2.0 / _SEQ)
    cosv = jnp.cos(ang)
    sinv = jnp.sin(ang)

    def tn(a, b):
        return _bf16dot(a, b, ((0,), (0,)))

    def nn(a, b):
        return _bf16dot(a, b, ((1,), (0,)))

    for h in range(_HEADS):
        sl = slice(h * _HD, (h + 1) * _HD)
        qnn = _nonneg(qh[:, sl])
        knn = _nonneg(kh[:, sl])
        vh_h = vh[:, sl]
        kcv = tn(knn * cosv, vh_h)
        ksv = tn(knn * sinv, vh_h)
        vals = nn(qnn * cosv, kcv) + nn(qnn * sinv, ksv)
        out_ref[0, :, sl] = vals * mask[:, h:h + 1]


@functools.lru_cache(maxsize=1)
def _gpad_const():
    import numpy as np
    f32 = jnp.float32
    with jax.ensure_compile_time_eval():
        g = jax.random.gumbel(jax.random.key(12345),
                              (_BATCH * _HEADS, _SEQ - 1), dtype=f32)
        gt = g.reshape(_BATCH, _HEADS, _SEQ - 1).transpose(0, 2, 1)
        gpad = jnp.concatenate(
            [jnp.full((_BATCH, 1, _HEADS), _BONUS, f32), gt], axis=1)
        return np.asarray(jax.device_get(gpad))


def kernel(q, k, v, Wq, bq, Wk, bk, Wv, bv):
    f32 = jnp.float32
    gpad = jnp.asarray(_gpad_const())

    bspec = lambda shp: pl.BlockSpec(shp, lambda b: (0,) * len(shp))
    out = pl.pallas_call(
        _body,
        grid=(_BATCH,),
        in_specs=[
            pl.BlockSpec((1, _SEQ, _EMBED), lambda b: (b, 0, 0)),
            pl.BlockSpec((1, _SEQ, _EMBED), lambda b: (b, 0, 0)),
            pl.BlockSpec((1, _SEQ, _EMBED), lambda b: (b, 0, 0)),
            bspec((_EMBED, _EMBED)),
            bspec((_EMBED, _EMBED)),
            bspec((_EMBED, _EMBED)),
            bspec((1, _EMBED)),
            bspec((1, _EMBED)),
            bspec((1, _EMBED)),
            pl.BlockSpec((1, _SEQ, _HEADS), lambda b: (b, 0, 0)),
        ],
        out_specs=pl.BlockSpec((1, _SEQ, _EMBED), lambda b: (b, 0, 0)),
        out_shape=jax.ShapeDtypeStruct((_BATCH, _SEQ, _EMBED), f32),
    )(q.astype(jnp.bfloat16), k.astype(jnp.bfloat16), v.astype(jnp.bfloat16),
      Wq.astype(jnp.bfloat16), Wk.astype(jnp.bfloat16),
      Wv.astype(jnp.bfloat16), bq.reshape(1, -1), bk.reshape(1, -1),
      bv.reshape(1, -1), gpad)
    return out


# speedup vs baseline: 10.3334x; 1.0255x over previous
import math
import functools

import jax
import jax.numpy as jnp
from jax.experimental import pallas as pl

_EMBED = 768
_HEADS = 12
_SEQ = 577
_HD = 64
_R1 = 404
_BATCH = 32
_BONUS = 1e4
_NSTEPS = 30

def _nonneg(x):
    return jnp.where(x < 0, jnp.exp(x), x + 1.0)


def _bf16dot(a, b, dims):
    return jax.lax.dot_general(a.astype(jnp.bfloat16), b.astype(jnp.bfloat16),
                               (dims, ((), ())),
                               preferred_element_type=jnp.float32)


def _bdot(a16, b16, dims):
    return jax.lax.dot_general(a16, b16, (dims, ((), ())),
                               preferred_element_type=jnp.float32)


def _body(q_ref, k_ref, v_ref, wq_ref, wk_ref, wv_ref, bq_ref, bk_ref,
          bv_ref, gp_ref, out_ref):
    f32 = jnp.float32
    bf16 = jnp.bfloat16
    qb = q_ref[0].astype(bf16)
    kb = k_ref[0].astype(bf16)
    vb = v_ref[0].astype(bf16)

    def proj(x16, w_ref, b_ref):
        return _bdot(x16, w_ref[...].astype(bf16), ((1,), (1,))) + b_ref[...]

    qh = proj(qb, wq_ref, bq_ref)
    kh = proj(kb, wk_ref, bk_ref)
    vh = proj(vb, wv_ref, bv_ref)

    kh16 = kh.astype(jnp.bfloat16).astype(f32)
    u16 = qh[0:1, :].astype(jnp.bfloat16).astype(f32)
    khu = kh16 * u16
    scores = jnp.concatenate(
        [jnp.sum(khu[:, h * _HD:(h + 1) * _HD], axis=1, keepdims=True)
         for h in range(_HEADS)], axis=1) * 0.125
    keys = scores[1:, :] + gp_ref[0, 1:, :]

    lo0 = jnp.min(keys, axis=0, keepdims=True) - 1.0
    hi0 = jnp.max(keys, axis=0, keepdims=True) + 1.0

    def step(_, carry):
        lo, hi = carry
        mid = 0.5 * (lo + hi)
        cnt = jnp.sum((keys >= mid).astype(f32), axis=0, keepdims=True)
        pred = cnt >= float(_R1 - 1)
        return jnp.where(pred, mid, lo), jnp.where(pred, hi, mid)

    lo, _ = jax.lax.fori_loop(0, _NSTEPS, step, (lo0, hi0))
    mask = jnp.concatenate(
        [jnp.ones((1, _HEADS), f32), (keys >= lo).astype(f32)], axis=0)

    ang = jax.lax.broadcasted_iota(jnp.int32, (_SEQ, 1), 0).astype(f32) * (
        math.pi / 2.0 / _SEQ)
    cosv = jnp.cos(ang)
    sinv = jnp.sin(ang)

    def tn(a, b):
        return _bf16dot(a, b, ((0,), (0,)))

    def nn(a, b):
        return _bf16dot(a, b, ((1,), (0,)))

    for h in range(_HEADS):
        sl = slice(h * _HD, (h + 1) * _HD)
        qnn = _nonneg(qh[:, sl])
        knn = _nonneg(kh[:, sl])
        vh_h = vh[:, sl]
        kcv = tn(knn * cosv, vh_h)
        ksv = tn(knn * sinv, vh_h)
        vals = nn(qnn * cosv, kcv) + nn(qnn * sinv, ksv)
        out_ref[0, :, sl] = vals * mask[:, h:h + 1]


@functools.lru_cache(maxsize=1)
def _gpad_const():
    import numpy as np
    f32 = jnp.float32
    with jax.ensure_compile_time_eval():
        g = jax.random.gumbel(jax.random.key(12345),
                              (_BATCH * _HEADS, _SEQ - 1), dtype=f32)
        gt = g.reshape(_BATCH, _HEADS, _SEQ - 1).transpose(0, 2, 1)
        gpad = jnp.concatenate(
            [jnp.full((_BATCH, 1, _HEADS), _BONUS, f32), gt], axis=1)
        return np.asarray(jax.device_get(gpad))


def kernel(q, k, v, Wq, bq, Wk, bk, Wv, bv):
    f32 = jnp.float32
    gpad = jnp.asarray(_gpad_const())

    bspec = lambda shp: pl.BlockSpec(shp, lambda b: (0,) * len(shp))
    out = pl.pallas_call(
        _body,
        grid=(_BATCH,),
        in_specs=[
            pl.BlockSpec((1, _SEQ, _EMBED), lambda b: (b, 0, 0)),
            pl.BlockSpec((1, _SEQ, _EMBED), lambda b: (b, 0, 0)),
            pl.BlockSpec((1, _SEQ, _EMBED), lambda b: (b, 0, 0)),
            bspec((_EMBED, _EMBED)),
            bspec((_EMBED, _EMBED)),
            bspec((_EMBED, _EMBED)),
            bspec((1, _EMBED)),
            bspec((1, _EMBED)),
            bspec((1, _EMBED)),
            pl.BlockSpec((1, _SEQ, _HEADS), lambda b: (b, 0, 0)),
        ],
        out_specs=pl.BlockSpec((1, _SEQ, _EMBED), lambda b: (b, 0, 0)),
        out_shape=jax.ShapeDtypeStruct((_BATCH, _SEQ, _EMBED), f32),
    )(q, k, v, Wq, Wk, Wv, bq.reshape(1, -1), bk.reshape(1, -1),
      bv.reshape(1, -1), gpad)
    return out
